# baseline (speedup 1.0000x reference)
import functools

import jax
import jax.numpy as jnp
from jax import lax
from jax.experimental import pallas as pl
from jax.experimental.pallas import tpu as pltpu
from jax.experimental.pallas import tpu_sc as plsc

_V = 100000
_E = 64
_B = 1024
_L = 16
_NC, _NS = 2, 16
_NW = _NC * _NS
_BPW = _B // _NW
_NIDX = _BPW * _E
_IDX_CHUNK = 128
_NDMA = _NIDX // _IDX_CHUNK


def _rsqrt(x):
    i = lax.bitcast_convert_type(x, jnp.int32)
    y = lax.bitcast_convert_type(jnp.int32(0x5F3759DF) - (i >> 1), jnp.float32)
    for _ in range(3):
        y = y * (1.5 - 0.5 * x * y * y)
    return y


def _body(x_hbm, w_hbm, b_hbm, out_hbm, idx_v, gath_v, x_v, b_v, out_v, sem):
    wid = lax.axis_index("s") * _NC + lax.axis_index("c")
    base = wid * _BPW

    pltpu.sync_copy(x_hbm.at[pl.ds(base, _BPW)], x_v)
    pltpu.sync_copy(b_hbm, b_v)

    xv0 = x_v[pl.ds(0, _L)]
    xv1 = x_v[pl.ds(_L, _L)]

    for e in range(_E):
        off = e * _V
        idx_v[pl.ds(e * _BPW, _L)] = xv0 + off
        idx_v[pl.ds(e * _BPW + _L, _L)] = xv1 + off

    copies = []
    for j in range(_NDMA):
        copies.append(
            pltpu.async_copy(
                w_hbm.at[idx_v.at[pl.ds(j * _IDX_CHUNK, _IDX_CHUNK)]],
                gath_v.at[pl.ds(j * _IDX_CHUNK, _IDX_CHUNK)],
                sem,
            )
        )
    for c in copies:
        c.wait()

    bv = [b_v[pl.ds(k * _L, _L)] for k in range(_E // _L)]

    acc0 = jnp.zeros((_L,), jnp.float32)
    acc1 = jnp.zeros((_L,), jnp.float32)
    for e in range(_E):
        be = bv[e // _L][e % _L]
        v0 = gath_v[pl.ds(e * _BPW, _L)] + be
        v1 = gath_v[pl.ds(e * _BPW + _L, _L)] + be
        acc0 = acc0 + v0 * v0
        acc1 = acc1 + v1 * v1

    r0 = _rsqrt(jnp.maximum(acc0, 1e-24))
    r1 = _rsqrt(jnp.maximum(acc1, 1e-24))

    iota0 = lax.iota(jnp.int32, _L)
    iota1 = iota0 + _L
    for e in range(_E):
        be = bv[e // _L][e % _L]
        ecol = jnp.full((_L,), e, jnp.int32)
        v0 = (gath_v[pl.ds(e * _BPW, _L)] + be) * r0
        v1 = (gath_v[pl.ds(e * _BPW + _L, _L)] + be) * r1
        plsc.store_scatter(out_v, [iota0, ecol], v0)
        plsc.store_scatter(out_v, [iota1, ecol], v1)

    pltpu.sync_copy(out_v, out_hbm.at[pl.ds(base, _BPW)])


@jax.jit
def _road2vec_sc(x, w_flat, b):
    mesh = plsc.VectorSubcoreMesh(core_axis_name="c", subcore_axis_name="s")
    return pl.kernel(
        _body,
        mesh=mesh,
        compiler_params=pltpu.CompilerParams(needs_layout_passes=False),
        out_type=jax.ShapeDtypeStruct((_B, _E), jnp.float32),
        scratch_types=[
            pltpu.VMEM((_NIDX,), jnp.int32),
            pltpu.VMEM((_NIDX,), jnp.float32),
            pltpu.VMEM((_BPW,), jnp.int32),
            pltpu.VMEM((_E,), jnp.float32),
            pltpu.VMEM((_BPW, _E), jnp.float32),
            pltpu.SemaphoreType.DMA,
        ],
    )(x, w_flat, b)


def kernel(x, W, b):
    return _road2vec_sc(x.astype(jnp.int32), W.reshape(-1), b)
